# Initial kernel scaffold; baseline (speedup 1.0000x reference)
import jax
import jax.numpy as jnp
from jax.experimental import pallas as pl


def kernel(input, adj, W):
    raise NotImplementedError("write your pallas kernel here")


# speedup vs baseline: 5.3548x; 5.3548x over previous
import functools

import jax
import jax.numpy as jnp
from jax import lax
from jax.experimental import pallas as pl
from jax.experimental.pallas import tpu as pltpu
from jax.experimental.pallas import tpu_sc as plsc

N = 10000
E = 320000
D = 128
DP = 144
SW = 16
ALPHA = 0.2
MIN_NORM = 1e-15
BALL_EPS = 4e-3

NC = 2
NS = 16
NW = NC * NS
EPT = E // NW
K = 80
NCHUNK = EPT // K
G = K // 16
NP = 10240
ROWS_PT = NP // NS

_LN2 = 0.6931471805599453


def _artanh(x):
    x = jnp.clip(x, -1.0 + 1e-7, 1.0 - 1e-7)
    return 0.5 * (jnp.log1p(x) - jnp.log1p(-x))


def _prep_body(x_ref, w_ref, hbf_ref, side_ref):
    x = x_ref[:, :]
    W = w_ref[:, :]
    rb = x.shape[0]
    x2 = jnp.sum(x * x, axis=1, keepdims=True)
    x_norm = jnp.maximum(jnp.sqrt(x2), MIN_NORM)
    mx = lax.dot_general(x, W, (((1,), (1,)), ((), ())),
                         preferred_element_type=jnp.float32)
    mx2 = jnp.sum(mx * mx, axis=1, keepdims=True)
    mx_norm = jnp.maximum(jnp.sqrt(mx2), MIN_NORM)
    res = jnp.tanh(mx_norm / x_norm * _artanh(x_norm)) * mx / mx_norm
    cond = jnp.all(mx == 0.0, axis=1, keepdims=True)
    res = jnp.where(cond, 0.0, res)
    r2 = jnp.sum(res * res, axis=1, keepdims=True)
    r_norm = jnp.maximum(jnp.sqrt(r2), MIN_NORM)
    maxnorm = 1.0 - BALL_EPS
    h = jnp.where(r_norm > maxnorm, res / r_norm * maxnorm, res)
    n = jnp.sum(h * h, axis=1, keepdims=True)
    pn = jnp.maximum(jnp.sqrt(n), MIN_NORM)
    s = _artanh(pn) / pn
    hbf_ref[:, :] = h.astype(jnp.bfloat16)
    side_ref[:, :] = jnp.concatenate(
        [n, s, jnp.zeros((rb, SW - 2), jnp.float32)], axis=1)


def _prep(x, W):
    RB = 2000
    return pl.pallas_call(
        _prep_body,
        grid=(N // RB,),
        in_specs=[
            pl.BlockSpec((RB, D), lambda i: (i, 0)),
            pl.BlockSpec((D, D), lambda i: (0, 0)),
        ],
        out_specs=[
            pl.BlockSpec((RB, D), lambda i: (i, 0)),
            pl.BlockSpec((RB, SW), lambda i: (i, 0)),
        ],
        out_shape=[
            jax.ShapeDtypeStruct((N, D), jnp.bfloat16),
            jax.ShapeDtypeStruct((N, SW), jnp.float32),
        ],
    )(x, W)


def _edge_body(hbf, side, srcv, dstv, out,
               ixs, ixd, rsb, rdb, ssb, sdb, wrow, part, acc, gsem):
    cid = lax.axis_index("c")
    sid = lax.axis_index("s")
    wid = sid * NC + cid
    zvec = jnp.zeros((16,), jnp.float32)
    iota = lax.iota(jnp.int32, 16)
    c0 = jnp.zeros((16,), jnp.int32)
    c1 = jnp.full((16,), 1, jnp.int32)
    c128 = jnp.full((16,), D, jnp.int32)
    e0 = 2 * iota
    e1 = e0 + 1

    def _z1(i, _):
        r = i // 9
        col = (i % 9) * 16
        wrow[r, pl.ds(col, 16)] = zvec
        return 0
    lax.fori_loop(0, K * 9, _z1, 0)

    row0 = sid * ROWS_PT

    def _zacc(q, _):
        pltpu.sync_copy(wrow, acc.at[pl.ds(row0 + q * K, K)])
        return 0
    lax.fori_loop(0, ROWS_PT // K, _zacc, 0)
    plsc.subcore_barrier()

    base = wid * EPT

    def _issue(ci, bs):
        off = base + ci * K
        pltpu.sync_copy(srcv.at[pl.ds(off, K)], ixs[bs])
        pltpu.sync_copy(dstv.at[pl.ds(off, K)], ixd[bs])
        pltpu.async_copy(hbf.at[ixs[bs]], rsb[bs], gsem[4 * bs + 0])
        pltpu.async_copy(hbf.at[ixd[bs]], rdb[bs], gsem[4 * bs + 1])
        pltpu.async_copy(side.at[ixs[bs]], ssb[bs], gsem[4 * bs + 2])
        pltpu.async_copy(side.at[ixd[bs]], sdb[bs], gsem[4 * bs + 3])

    def _wait(bs):
        pltpu.make_async_copy(hbf.at[ixs[bs]], rsb[bs], gsem[4 * bs + 0]).wait()
        pltpu.make_async_copy(hbf.at[ixd[bs]], rdb[bs], gsem[4 * bs + 1]).wait()
        pltpu.make_async_copy(side.at[ixs[bs]], ssb[bs], gsem[4 * bs + 2]).wait()
        pltpu.make_async_copy(side.at[ixd[bs]], sdb[bs], gsem[4 * bs + 3]).wait()

    def _compute(bs):
        rs, rd, ss, sd = rsb[bs], rdb[bs], ssb[bs], sdb[bs]

        def _group(gg, _):
            for j in range(16):
                r = gg * 16 + j
                prod = rs[r, pl.ds(0, 32)] * rd[r, pl.ds(0, 32)]
                pa, pb = plsc.unpack(prod, format=plsc.PackFormat.INTERLEAVED,
                                     preferred_element_type=jnp.float32)
                accv = pa + pb
                for b in range(1, 4):
                    prod = (rs[r, pl.ds(b * 32, 32)]
                            * rd[r, pl.ds(b * 32, 32)])
                    pa, pb = plsc.unpack(
                        prod, format=plsc.PackFormat.INTERLEAVED,
                        preferred_element_type=jnp.float32)
                    accv = accv + pa + pb
                plsc.store_scatter(
                    part, [iota, jnp.full((16,), j, jnp.int32)], accv)
            dots = part[0, :]
            for l in range(1, 16):
                dots = dots + part[l, :]
            ii = gg * 16 + iota
            n_s = plsc.load_gather(ss, [ii, c0])
            n_d = plsc.load_gather(sd, [ii, c0])
            s_d = plsc.load_gather(sd, [ii, c1])
            A = 1.0 - 2.0 * dots + n_d
            B = 1.0 - n_s
            num2 = A * A * n_s + B * B * n_d - 2.0 * A * B * dots
            den = jnp.maximum(1.0 - 2.0 * dots + n_s * n_d, MIN_NORM)
            xx = jnp.maximum(num2, 1e-30)
            yi = 0x5F3759DF - (plsc.bitcast(xx, jnp.int32) >> 1)
            y = plsc.bitcast(yi, jnp.float32)
            y = y * (1.5 - 0.5 * xx * y * y)
            y = y * (1.5 - 0.5 * xx * y * y)
            y = y * (1.5 - 0.5 * xx * y * y)
            nrm = (xx * y) / den
            z = jnp.minimum(nrm, 1.0 - 1e-7)
            rr = (1.0 + z) / (1.0 - z)
            ri = plsc.bitcast(rr, jnp.int32)
            ex = (ri >> 23) - 127
            m = plsc.bitcast((ri & 0x007FFFFF) | 0x3F800000, jnp.float32)
            big = m > 1.4142135623730951
            m = jnp.where(big, 0.5 * m, m)
            ex = ex + big.astype(jnp.int32)
            t = (m - 1.0) / (m + 1.0)
            t2 = t * t
            pp = (1.0 / 9.0)
            pp = pp * t2 + (1.0 / 7.0)
            pp = pp * t2 + (1.0 / 5.0)
            pp = pp * t2 + (1.0 / 3.0)
            pp = pp * t2 + 1.0
            logr = ex.astype(jnp.float32) * _LN2 + 2.0 * t * pp
            at = 0.5 * logr
            w = jnp.exp((4.0 * ALPHA) * at * at)
            plsc.store_scatter(wrow, [ii, c128], w)
            ws = w * s_d
            for j in range(16):
                r = gg * 16 + j
                wsj = ws[j]
                rfull = jnp.full((16,), 1, jnp.int32) * r
                for b in range(4):
                    v = rd[r, pl.ds(b * 32, 32)]
                    pa, pb = plsc.unpack(
                        v, format=plsc.PackFormat.INTERLEAVED,
                        preferred_element_type=jnp.float32)
                    plsc.store_scatter(wrow, [rfull, 32 * b + e0], pa * wsj)
                    plsc.store_scatter(wrow, [rfull, 32 * b + e1], pb * wsj)
            return 0

        lax.fori_loop(0, G, _group, 0)

    def _do_chunk(bs, prefetch_ci, guard):
        _wait(bs)
        _compute(bs)
        pltpu.sync_copy(wrow, acc.at[ixs[bs]], add=True)
        if guard is None:
            _issue(prefetch_ci, bs)
        elif guard:
            @pl.when(prefetch_ci < NCHUNK)
            def _():
                _issue(prefetch_ci, bs)

    _issue(0, 0)
    _issue(1, 1)

    def _pair(g2, _):
        a = 2 * g2
        _do_chunk(0, a + 2, None)
        _do_chunk(1, a + 3, True)
        return 0
    lax.fori_loop(0, (NCHUNK - 1) // 2, _pair, 0)
    _do_chunk(0, 0, False)

    plsc.subcore_barrier()

    def _wout(q, _):
        r0 = row0 + q * 128
        pltpu.sync_copy(acc.at[pl.ds(r0, 128)], out.at[cid, pl.ds(r0, 128)])
        return 0
    lax.fori_loop(0, ROWS_PT // 128, _wout, 0)


_edge_kernel = functools.partial(
    pl.kernel,
    out_type=jax.ShapeDtypeStruct((NC, NP, DP), jnp.float32),
    mesh=plsc.VectorSubcoreMesh(core_axis_name="c", subcore_axis_name="s"),
    compiler_params=pltpu.CompilerParams(
        needs_layout_passes=False, use_tc_tiling_on_sc=False),
    scratch_types=[
        [pltpu.VMEM((K,), jnp.int32)] * 2,
        [pltpu.VMEM((K,), jnp.int32)] * 2,
        [pltpu.VMEM((K, D), jnp.bfloat16)] * 2,
        [pltpu.VMEM((K, D), jnp.bfloat16)] * 2,
        [pltpu.VMEM((K, SW), jnp.float32)] * 2,
        [pltpu.VMEM((K, SW), jnp.float32)] * 2,
        pltpu.VMEM((K, DP), jnp.float32),
        pltpu.VMEM((16, 16), jnp.float32),
        pltpu.VMEM_SHARED((NP, DP), jnp.float32),
        [pltpu.SemaphoreType.DMA] * 8,
    ],
)(_edge_body)


def _final_body(p_ref, o_ref):
    acc = p_ref[0] + p_ref[1]
    hp = acc[:, 0:D] / acc[:, D:D + 1]
    hp = jnp.maximum(hp, 0.0)
    u2 = jnp.sum(hp * hp, axis=1, keepdims=True)
    u_norm = jnp.maximum(jnp.sqrt(u2), MIN_NORM)
    res = jnp.tanh(u_norm) * hp / u_norm
    r2 = jnp.sum(res * res, axis=1, keepdims=True)
    r_norm = jnp.maximum(jnp.sqrt(r2), MIN_NORM)
    maxnorm = 1.0 - BALL_EPS
    o_ref[:, :] = jnp.where(r_norm > maxnorm, res / r_norm * maxnorm, res)


def _final(parts):
    RB = 2000
    return pl.pallas_call(
        _final_body,
        grid=(N // RB,),
        in_specs=[pl.BlockSpec((NC, RB, DP), lambda i: (0, i, 0))],
        out_specs=pl.BlockSpec((RB, D), lambda i: (i, 0)),
        out_shape=jax.ShapeDtypeStruct((N, D), jnp.float32),
    )(parts)


def kernel(input, adj, W):
    hbf, side = _prep(input, W)
    parts = _edge_kernel(hbf, side, adj[0], adj[1])
    return _final(parts)
